# baseline (speedup 1.0000x reference)
import jax
import jax.numpy as jnp
from jax.experimental import pallas as pl
from jax.experimental.pallas import tpu as pltpu

_BM = 400
_BMB = 1024

_COMPILER_PARAMS = pltpu.CompilerParams(
    dimension_semantics=("arbitrary",),
    vmem_limit_bytes=100 * 1024 * 1024,
)


def _layer1_kernel(x_ref, w1_ref, adj_ref, b_ref, w_ref, out_ref, adj16_ref,
                   y1_scr):
    @pl.when(pl.program_id(0) == 0)
    def _():
        y1_scr[...] = jnp.dot(
            x_ref[...].astype(jnp.bfloat16), w1_ref[...],
            preferred_element_type=jnp.float32,
        ).astype(jnp.bfloat16)

    a16 = adj_ref[...].astype(jnp.bfloat16)
    adj16_ref[...] = a16
    t = jnp.dot(a16, y1_scr[...], preferred_element_type=jnp.float32)
    t = jnp.maximum(t + b_ref[...], 0.0)
    out_ref[...] = jnp.dot(
        t.astype(jnp.bfloat16), w_ref[...],
        preferred_element_type=jnp.float32,
    ).astype(jnp.bfloat16)


def _layer1(x, w1, adj, b, w):
    n, d = adj.shape[0], w1.shape[1]
    return pl.pallas_call(
        _layer1_kernel,
        grid=(pl.cdiv(n, _BM),),
        in_specs=[
            pl.BlockSpec((n, x.shape[1]), lambda i: (0, 0)),
            pl.BlockSpec((w1.shape[0], d), lambda i: (0, 0)),
            pl.BlockSpec((_BM, n), lambda i: (i, 0)),
            pl.BlockSpec((1, d), lambda i: (0, 0)),
            pl.BlockSpec((d, w.shape[1]), lambda i: (0, 0)),
        ],
        out_specs=[
            pl.BlockSpec((_BM, w.shape[1]), lambda i: (i, 0)),
            pl.BlockSpec((_BM, n), lambda i: (i, 0)),
        ],
        out_shape=[
            jax.ShapeDtypeStruct((n, w.shape[1]), jnp.bfloat16),
            jax.ShapeDtypeStruct((n, n), jnp.bfloat16),
        ],
        scratch_shapes=[pltpu.VMEM((n, d), jnp.bfloat16)],
        compiler_params=_COMPILER_PARAMS,
    )(x, w1, adj, b, w)


def _tail_kernel(adj16_ref, y2_ref, ba_ref, b2_ref, w2_ref, out_ref, y3_scr):
    i = pl.program_id(1)
    n = y2_ref.shape[0]

    @pl.when(pl.program_id(0) == 0)
    def _():
        t = jnp.dot(adj16_ref[...], y2_ref[...],
                    preferred_element_type=jnp.float32)
        t = jnp.maximum(t + ba_ref[...], 0.0)
        y3_scr[pl.ds(i * _BMB, _BMB), :] = jnp.dot(
            t.astype(jnp.bfloat16), w2_ref[...],
            preferred_element_type=jnp.float32,
        ).astype(jnp.bfloat16)

    @pl.when(pl.program_id(0) == 1)
    def _():
        t = jnp.dot(adj16_ref[...], y3_scr[0:n, :],
                    preferred_element_type=jnp.float32)
        out_ref[...] = jnp.maximum(t + b2_ref[...], 0.0)


def _tail(adj16, y2, ba, b2, w2):
    n, d = adj16.shape[0], y2.shape[1]
    nblk = pl.cdiv(n, _BMB)
    return pl.pallas_call(
        _tail_kernel,
        grid=(2, nblk),
        in_specs=[
            pl.BlockSpec((_BMB, n), lambda p, i: (i, 0)),
            pl.BlockSpec((n, d), lambda p, i: (0, 0)),
            pl.BlockSpec((1, d), lambda p, i: (0, 0)),
            pl.BlockSpec((1, d), lambda p, i: (0, 0)),
            pl.BlockSpec((d, d), lambda p, i: (0, 0)),
        ],
        out_specs=pl.BlockSpec((_BMB, d), lambda p, i: (p * i, 0)),
        out_shape=jax.ShapeDtypeStruct((n, d), jnp.float32),
        scratch_shapes=[pltpu.VMEM((nblk * _BMB, d), jnp.bfloat16)],
        compiler_params=pltpu.CompilerParams(
            dimension_semantics=("arbitrary", "arbitrary"),
            vmem_limit_bytes=100 * 1024 * 1024,
        ),
    )(adj16, y2, ba, b2, w2)


def kernel(adj, inputs, W1, b1, Wa, ba, W2, b2):
    w1 = W1.astype(jnp.bfloat16)
    wa = Wa.astype(jnp.bfloat16)
    w2 = W2.astype(jnp.bfloat16)
    b1r = b1.reshape(1, -1)
    bar = ba.reshape(1, -1)
    b2r = b2.reshape(1, -1)

    y2, adj16 = _layer1(inputs, w1, adj, b1r, wa)
    return _tail(adj16, y2, bar, b2r, w2)


# speedup vs baseline: 1.0073x; 1.0073x over previous
import jax
import jax.numpy as jnp
from jax.experimental import pallas as pl
from jax.experimental.pallas import tpu as pltpu

_BM = 448
_BMB = 1280

_COMPILER_PARAMS = pltpu.CompilerParams(
    dimension_semantics=("arbitrary",),
    vmem_limit_bytes=100 * 1024 * 1024,
)


def _layer1_kernel(x_ref, w1_ref, adj_ref, b_ref, w_ref, out_ref, adj16_ref,
                   y1_scr):
    @pl.when(pl.program_id(0) == 0)
    def _():
        y1_scr[...] = jnp.dot(
            x_ref[...].astype(jnp.bfloat16), w1_ref[...],
            preferred_element_type=jnp.float32,
        ).astype(jnp.bfloat16)

    a16 = adj_ref[...].astype(jnp.bfloat16)
    adj16_ref[...] = a16
    t = jnp.dot(a16, y1_scr[...], preferred_element_type=jnp.float32)
    t = jnp.maximum(t + b_ref[...], 0.0)
    out_ref[...] = jnp.dot(
        t.astype(jnp.bfloat16), w_ref[...],
        preferred_element_type=jnp.float32,
    ).astype(jnp.bfloat16)


def _layer1(x, w1, adj, b, w):
    n, d = adj.shape[0], w1.shape[1]
    return pl.pallas_call(
        _layer1_kernel,
        grid=(pl.cdiv(n, _BM),),
        in_specs=[
            pl.BlockSpec((n, x.shape[1]), lambda i: (0, 0)),
            pl.BlockSpec((w1.shape[0], d), lambda i: (0, 0)),
            pl.BlockSpec((_BM, n), lambda i: (i, 0)),
            pl.BlockSpec((1, d), lambda i: (0, 0)),
            pl.BlockSpec((d, w.shape[1]), lambda i: (0, 0)),
        ],
        out_specs=[
            pl.BlockSpec((_BM, w.shape[1]), lambda i: (i, 0)),
            pl.BlockSpec((_BM, n), lambda i: (i, 0)),
        ],
        out_shape=[
            jax.ShapeDtypeStruct((n, w.shape[1]), jnp.bfloat16),
            jax.ShapeDtypeStruct((n, n), jnp.bfloat16),
        ],
        scratch_shapes=[pltpu.VMEM((n, d), jnp.bfloat16)],
        compiler_params=_COMPILER_PARAMS,
    )(x, w1, adj, b, w)


def _tail_kernel(adj16_ref, y2_ref, ba_ref, b2_ref, w2_ref, out_ref, y3_scr):
    i = pl.program_id(1)
    n = y2_ref.shape[0]

    @pl.when(pl.program_id(0) == 0)
    def _():
        t = jnp.dot(adj16_ref[...], y2_ref[...],
                    preferred_element_type=jnp.float32)
        t = jnp.maximum(t + ba_ref[...], 0.0)
        y3_scr[pl.ds(i * _BMB, _BMB), :] = jnp.dot(
            t.astype(jnp.bfloat16), w2_ref[...],
            preferred_element_type=jnp.float32,
        ).astype(jnp.bfloat16)

    @pl.when(pl.program_id(0) == 1)
    def _():
        t = jnp.dot(adj16_ref[...], y3_scr[0:n, :],
                    preferred_element_type=jnp.float32)
        out_ref[...] = jnp.maximum(t + b2_ref[...], 0.0)


def _tail(adj16, y2, ba, b2, w2):
    n, d = adj16.shape[0], y2.shape[1]
    nblk = pl.cdiv(n, _BMB)
    return pl.pallas_call(
        _tail_kernel,
        grid=(2, nblk),
        in_specs=[
            pl.BlockSpec((_BMB, n), lambda p, i: (i, 0)),
            pl.BlockSpec((n, d), lambda p, i: (0, 0)),
            pl.BlockSpec((1, d), lambda p, i: (0, 0)),
            pl.BlockSpec((1, d), lambda p, i: (0, 0)),
            pl.BlockSpec((d, d), lambda p, i: (0, 0)),
        ],
        out_specs=pl.BlockSpec((_BMB, d), lambda p, i: (p * i, 0)),
        out_shape=jax.ShapeDtypeStruct((n, d), jnp.float32),
        scratch_shapes=[pltpu.VMEM((nblk * _BMB, d), jnp.bfloat16)],
        compiler_params=pltpu.CompilerParams(
            dimension_semantics=("arbitrary", "arbitrary"),
            vmem_limit_bytes=100 * 1024 * 1024,
        ),
    )(adj16, y2, ba, b2, w2)


def kernel(adj, inputs, W1, b1, Wa, ba, W2, b2):
    w1 = W1.astype(jnp.bfloat16)
    wa = Wa.astype(jnp.bfloat16)
    w2 = W2.astype(jnp.bfloat16)
    b1r = b1.reshape(1, -1)
    bar = ba.reshape(1, -1)
    b2r = b2.reshape(1, -1)

    y2, adj16 = _layer1(inputs, w1, adj, b1r, wa)
    return _tail(adj16, y2, bar, b2r, w2)
